# baseline (speedup 1.0000x reference)
import functools

import jax
import jax.numpy as jnp
from jax import lax
from jax.experimental import pallas as pl
from jax.experimental.pallas import tpu as pltpu
from jax.experimental.pallas import tpu_sc as plsc

N = 10000
E = 320000
D_IN = 128
D_HID = 128
D_OUT = 64

NP = 10240
EP = 327680
NC = 2
NS = 16
NW = NC * NS
RPT = EP // (NW * 128)
STRIPE = NP // NS

_MESH = plsc.VectorSubcoreMesh(core_axis_name="c", subcore_axis_name="s")


@functools.partial(
    pl.kernel,
    mesh=_MESH,
    compiler_params=pltpu.CompilerParams(use_tc_tiling_on_sc=False),
    out_type=[jax.ShapeDtypeStruct((NP,), jnp.float32) for _ in range(4)],
    scratch_types=[
        pltpu.VMEM((RPT, 128), jnp.int32),
        pltpu.VMEM((RPT, 128), jnp.int32),
        pltpu.VMEM((128,), jnp.float32),
        pltpu.VMEM_SHARED((NP,), jnp.float32),
        pltpu.VMEM_SHARED((NP,), jnp.float32),
        pltpu.SemaphoreType.DMA,
    ],
)
def _sc_degrees(srcw, dstw, ones_h, zeros_h, ds0, ds1, dd0, dd1,
                src_v, dst_v, ones_v, dout_sh, din_sh, sem):
    cid = lax.axis_index("c")
    sid = lax.axis_index("s")
    wid = sid * NC + cid
    r0 = sid * STRIPE
    pltpu.sync_copy(zeros_h, dout_sh.at[pl.ds(r0, STRIPE)])
    pltpu.sync_copy(zeros_h, din_sh.at[pl.ds(r0, STRIPE)])
    pltpu.sync_copy(ones_h, ones_v)
    pltpu.sync_copy(srcw.at[wid], src_v)
    pltpu.sync_copy(dstw.at[wid], dst_v)
    plsc.subcore_barrier()

    def body(j, carry):
        pltpu.async_copy(ones_v, dout_sh.at[src_v.at[j]], sem, add=True)
        pltpu.async_copy(ones_v, din_sh.at[dst_v.at[j]], sem, add=True)
        return carry

    lax.fori_loop(0, RPT, body, 0)

    def drain(j, carry):
        pltpu.make_async_copy(ones_h, ones_v, sem).wait()
        pltpu.make_async_copy(ones_h, ones_v, sem).wait()
        return carry

    lax.fori_loop(0, RPT, drain, 0)
    plsc.subcore_barrier()

    @pl.when(cid == 0)
    def _():
        pltpu.sync_copy(dout_sh.at[pl.ds(r0, STRIPE)], ds0.at[pl.ds(r0, STRIPE)])
        pltpu.sync_copy(din_sh.at[pl.ds(r0, STRIPE)], dd0.at[pl.ds(r0, STRIPE)])

    @pl.when(cid == 1)
    def _():
        pltpu.sync_copy(dout_sh.at[pl.ds(r0, STRIPE)], ds1.at[pl.ds(r0, STRIPE)])
        pltpu.sync_copy(din_sh.at[pl.ds(r0, STRIPE)], dd1.at[pl.ds(r0, STRIPE)])


K = 4
NIT = RPT // (2 * K)


def _agg_pipeline(table, acc_sh, src_v, dst_v, buf_a, buf_b, sem_g, sem_s):
    for t in range(K):
        pltpu.async_copy(table.at[src_v.at[t]], buf_a.at[t], sem_g)

    def body(gg, carry):
        b0 = gg * 2 * K
        b1 = b0 + K
        for t in range(K):
            pltpu.make_async_copy(table.at[pl.ds(0, 128)], buf_a.at[t],
                                  sem_g).wait()
        s_a = [
            pltpu.async_copy(buf_a.at[t], acc_sh.at[dst_v.at[b0 + t]],
                             sem_s, add=True)
            for t in range(K)
        ]
        g_b = [
            pltpu.async_copy(table.at[src_v.at[b1 + t]], buf_b.at[t], sem_g)
            for t in range(K)
        ]
        for d in s_a:
            d.wait()
        for d in g_b:
            d.wait()
        s_b = [
            pltpu.async_copy(buf_b.at[t], acc_sh.at[dst_v.at[b1 + t]],
                             sem_s, add=True)
            for t in range(K)
        ]

        @pl.when(gg < NIT - 1)
        def _():
            for t in range(K):
                pltpu.async_copy(table.at[src_v.at[b1 + K + t]],
                                 buf_a.at[t], sem_g)

        for d in s_b:
            d.wait()
        return carry

    lax.fori_loop(0, NIT, body, 0)


def _make_sc_agg():
    feat = D_OUT

    @functools.partial(
        pl.kernel,
        mesh=_MESH,
        compiler_params=pltpu.CompilerParams(use_tc_tiling_on_sc=False),
        out_type=[jax.ShapeDtypeStruct((NP, feat), jnp.float32) for _ in range(2)],
        scratch_types=[
            pltpu.VMEM((RPT, 128), jnp.int32),
            pltpu.VMEM((RPT, 128), jnp.int32),
            pltpu.VMEM((K, 128, feat), jnp.float32),
            pltpu.VMEM((K, 128, feat), jnp.float32),
            pltpu.VMEM_SHARED((NP, feat), jnp.float32),
            pltpu.SemaphoreType.DMA,
            pltpu.SemaphoreType.DMA,
        ],
    )
    def agg(table, srcw, dstw, zeros_h, p0, p1,
            src_v, dst_v, buf_a, buf_b, acc_sh, sem_g, sem_s):
        cid = lax.axis_index("c")
        sid = lax.axis_index("s")
        wid = sid * NC + cid
        r0 = sid * STRIPE
        pltpu.sync_copy(zeros_h, acc_sh.at[pl.ds(r0, STRIPE)])
        pltpu.sync_copy(srcw.at[wid], src_v)
        pltpu.sync_copy(dstw.at[wid], dst_v)
        plsc.subcore_barrier()
        _agg_pipeline(table, acc_sh, src_v, dst_v, buf_a, buf_b, sem_g, sem_s)
        plsc.subcore_barrier()

        @pl.when(cid == 0)
        def _():
            pltpu.sync_copy(acc_sh.at[pl.ds(r0, STRIPE)], p0.at[pl.ds(r0, STRIPE)])

        @pl.when(cid == 1)
        def _():
            pltpu.sync_copy(acc_sh.at[pl.ds(r0, STRIPE)], p1.at[pl.ds(r0, STRIPE)])

    return agg


_sc_agg64 = _make_sc_agg()


@functools.partial(
    pl.kernel,
    mesh=_MESH,
    compiler_params=pltpu.CompilerParams(use_tc_tiling_on_sc=False),
    out_type=[jax.ShapeDtypeStruct((NP, D_OUT), jnp.float32) for _ in range(2)],
    scratch_types=[
        pltpu.VMEM((RPT, 128), jnp.int32),
        pltpu.VMEM((RPT, 128), jnp.int32),
        pltpu.VMEM((K, 128, D_OUT), jnp.float32),
        pltpu.VMEM((K, 128, D_OUT), jnp.float32),
        pltpu.VMEM_SHARED((NP, D_OUT), jnp.float32),
        pltpu.SemaphoreType.DMA,
        pltpu.SemaphoreType.DMA,
    ],
)
def _sc_agg_dual(table_l, table_r, srcw, dstw, zeros_h, p_l, p_r,
                 src_v, dst_v, buf_a, buf_b, acc_sh, sem_g, sem_s):
    cid = lax.axis_index("c")
    sid = lax.axis_index("s")
    r0 = sid * STRIPE

    def half(table, pout):
        pltpu.sync_copy(zeros_h, acc_sh.at[pl.ds(r0, STRIPE)])
        plsc.subcore_barrier()
        for p in range(2):
            pltpu.sync_copy(srcw.at[sid * 2 + p], src_v)
            pltpu.sync_copy(dstw.at[sid * 2 + p], dst_v)
            _agg_pipeline(table, acc_sh, src_v, dst_v, buf_a, buf_b,
                          sem_g, sem_s)
        plsc.subcore_barrier()
        pltpu.sync_copy(acc_sh.at[pl.ds(r0, STRIPE)], pout.at[pl.ds(r0, STRIPE)])

    @pl.when(cid == 0)
    def _():
        half(table_l, p_l)

    @pl.when(cid == 1)
    def _():
        half(table_r, p_r)


BR = 2048
BG = BR // 128
NB = NP // BR


def _norm4(a_ref, b_ref):
    return lax.rsqrt(jnp.maximum(a_ref[0] + b_ref[0], 1.0))


def _mm1_body(x_ref, w1_ref, yl_ref, yr_ref):
    y = jnp.dot(x_ref[...], w1_ref[...], preferred_element_type=jnp.float32)
    yl_ref[...] = y[:, :64]
    yr_ref[...] = y[:, 64:]


_mm1 = pl.pallas_call(
    _mm1_body,
    grid=(NB,),
    in_specs=[
        pl.BlockSpec((BR, D_IN), lambda i: (i, 0)),
        pl.BlockSpec((D_IN, D_HID), lambda i: (0, 0)),
    ],
    out_specs=[
        pl.BlockSpec((BR, 64), lambda i: (i, 0)),
        pl.BlockSpec((BR, 64), lambda i: (i, 0)),
    ],
    out_shape=[
        jax.ShapeDtypeStruct((NP, 64), jnp.float32),
        jax.ShapeDtypeStruct((NP, 64), jnp.float32),
    ],
)


def _scale_body(ul_ref, ur_ref, ds0_ref, ds1_ref, yl_ref, yr_ref):
    ns = _norm4(ds0_ref, ds1_ref)
    ns3 = ns[:, :, None]
    yl_ref[...] = jnp.reshape(
        jnp.reshape(ul_ref[...], (BG, 128, 64)) * ns3, (BR, 64))
    yr_ref[...] = jnp.reshape(
        jnp.reshape(ur_ref[...], (BG, 128, 64)) * ns3, (BR, 64))


_scale = pl.pallas_call(
    _scale_body,
    grid=(NB,),
    in_specs=[
        pl.BlockSpec((BR, 64), lambda i: (i, 0)),
        pl.BlockSpec((BR, 64), lambda i: (i, 0)),
        pl.BlockSpec((1, BG, 128), lambda i: (i, 0, 0)),
        pl.BlockSpec((1, BG, 128), lambda i: (i, 0, 0)),
    ],
    out_specs=[
        pl.BlockSpec((BR, 64), lambda i: (i, 0)),
        pl.BlockSpec((BR, 64), lambda i: (i, 0)),
    ],
    out_shape=[
        jax.ShapeDtypeStruct((NP, 64), jnp.float32),
        jax.ShapeDtypeStruct((NP, 64), jnp.float32),
    ],
)


def _mid_body(pl_ref, pr_ref, dd0_ref, dd1_ref,
              ds0_ref, ds1_ref, b1_ref, w2_ref, y_ref):
    nd = _norm4(dd0_ref, dd1_ref)
    ns = _norm4(ds0_ref, ds1_ref)
    agg2 = jnp.concatenate([pl_ref[...], pr_ref[...]], axis=-1)
    agg = jnp.reshape(agg2, (BG, 128, D_HID))
    h = jnp.maximum(agg * nd[:, :, None] + b1_ref[...][None, None, :], 0.0)
    hs = jnp.reshape(h * ns[:, :, None], (BR, D_HID))
    y_ref[...] = jnp.dot(hs, w2_ref[...], preferred_element_type=jnp.float32)


_mid = pl.pallas_call(
    _mid_body,
    grid=(NB,),
    in_specs=[
        pl.BlockSpec((BR, 64), lambda i: (i, 0)),
        pl.BlockSpec((BR, 64), lambda i: (i, 0)),
        pl.BlockSpec((1, BG, 128), lambda i: (i, 0, 0)),
        pl.BlockSpec((1, BG, 128), lambda i: (i, 0, 0)),
        pl.BlockSpec((1, BG, 128), lambda i: (i, 0, 0)),
        pl.BlockSpec((1, BG, 128), lambda i: (i, 0, 0)),
        pl.BlockSpec((D_HID,), lambda i: (0,)),
        pl.BlockSpec((D_HID, D_OUT), lambda i: (0, 0)),
    ],
    out_specs=pl.BlockSpec((BR, D_OUT), lambda i: (i, 0)),
    out_shape=jax.ShapeDtypeStruct((NP, D_OUT), jnp.float32),
)


def _final_body(p0_ref, p1_ref, dd0_ref, dd1_ref, b2_ref, o_ref):
    nd = _norm4(dd0_ref, dd1_ref)
    agg = jnp.reshape(p0_ref[...] + p1_ref[...], (BG, 128, D_OUT))
    z = agg * nd[:, :, None] + b2_ref[...][None, None, :]
    z = z - jnp.max(z, axis=-1, keepdims=True)
    ez = jnp.exp(z)
    sm = ez / jnp.sum(ez, axis=-1, keepdims=True)
    o_ref[...] = jnp.reshape(sm, (BR, D_OUT))


_final = pl.pallas_call(
    _final_body,
    grid=(NB,),
    in_specs=[
        pl.BlockSpec((BR, D_OUT), lambda i: (i, 0)),
        pl.BlockSpec((BR, D_OUT), lambda i: (i, 0)),
        pl.BlockSpec((1, BG, 128), lambda i: (i, 0, 0)),
        pl.BlockSpec((1, BG, 128), lambda i: (i, 0, 0)),
        pl.BlockSpec((D_OUT,), lambda i: (0,)),
    ],
    out_specs=pl.BlockSpec((BR, D_OUT), lambda i: (i, 0)),
    out_shape=jax.ShapeDtypeStruct((N, D_OUT), jnp.float32),
)


def kernel(x, edge_index, W1, b1, W2, b2):
    src = edge_index[0].astype(jnp.int32)
    dst = edge_index[1].astype(jnp.int32)
    pad = jnp.arange(EP - E, dtype=jnp.int32) % (NP - N) + N
    srcw = jnp.concatenate([src, pad]).reshape(NW, RPT, 128)
    dstw = jnp.concatenate([dst, pad]).reshape(NW, RPT, 128)

    ones1 = jnp.ones((128,), jnp.float32)
    zeros1 = jnp.zeros((STRIPE,), jnp.float32)
    zO = jnp.zeros((STRIPE, D_OUT), jnp.float32)

    ds0, ds1, dd0, dd1 = _sc_degrees(srcw, dstw, ones1, zeros1)
    ds0 = ds0.reshape(NB, BG, 128)
    ds1 = ds1.reshape(NB, BG, 128)
    dd0 = dd0.reshape(NB, BG, 128)
    dd1 = dd1.reshape(NB, BG, 128)

    ul, ur = _mm1(x, W1)
    y1l, y1r = _scale(ul, ur, ds0, ds1)
    al, ar = _sc_agg_dual(y1l, y1r, srcw, dstw, zO)
    y2 = _mid(al, ar, dd0, dd1, ds0, ds1, b1, W2)
    g0, g1 = _sc_agg64(y2, srcw, dstw, zO)
    return _final(g0, g1, dd0, dd1, b2)


# speedup vs baseline: 1.0353x; 1.0353x over previous
import functools

import jax
import jax.numpy as jnp
from jax import lax
from jax.experimental import pallas as pl
from jax.experimental.pallas import tpu as pltpu
from jax.experimental.pallas import tpu_sc as plsc

N = 10000
E = 320000
D_IN = 128
D_HID = 128
D_OUT = 64

NP = 10240
EP = 327680
NC = 2
NS = 16
NW = NC * NS
RPT = EP // (NW * 128)
NROW = E // 128
RREM = NROW - (NW - 1) * RPT
RPAD = RPT - RREM
STRIPE = NP // NS

_MESH = plsc.VectorSubcoreMesh(core_axis_name="c", subcore_axis_name="s")


def _load_slabs(ei, padw, w, src_v, dst_v):
    @pl.when(w < NW - 1)
    def _():
        pltpu.sync_copy(ei.at[0, pl.ds(w * RPT, RPT)], src_v)
        pltpu.sync_copy(ei.at[1, pl.ds(w * RPT, RPT)], dst_v)

    @pl.when(w == NW - 1)
    def _():
        pltpu.sync_copy(ei.at[0, pl.ds((NW - 1) * RPT, RREM)],
                        src_v.at[pl.ds(0, RREM)])
        pltpu.sync_copy(ei.at[1, pl.ds((NW - 1) * RPT, RREM)],
                        dst_v.at[pl.ds(0, RREM)])
        pltpu.sync_copy(padw, src_v.at[pl.ds(RREM, RPAD)])
        pltpu.sync_copy(padw, dst_v.at[pl.ds(RREM, RPAD)])


@functools.partial(
    pl.kernel,
    mesh=_MESH,
    compiler_params=pltpu.CompilerParams(use_tc_tiling_on_sc=False),
    out_type=[jax.ShapeDtypeStruct((NP,), jnp.float32) for _ in range(4)],
    scratch_types=[
        pltpu.VMEM((RPT, 128), jnp.int32),
        pltpu.VMEM((RPT, 128), jnp.int32),
        pltpu.VMEM((128,), jnp.float32),
        pltpu.VMEM_SHARED((NP,), jnp.float32),
        pltpu.VMEM_SHARED((NP,), jnp.float32),
        pltpu.SemaphoreType.DMA,
    ],
)
def _sc_degrees(ei, padw, ones_h, zeros_h, ds0, ds1, dd0, dd1,
                src_v, dst_v, ones_v, dout_sh, din_sh, sem):
    cid = lax.axis_index("c")
    sid = lax.axis_index("s")
    wid = sid * NC + cid
    r0 = sid * STRIPE
    pltpu.sync_copy(zeros_h, dout_sh.at[pl.ds(r0, STRIPE)])
    pltpu.sync_copy(zeros_h, din_sh.at[pl.ds(r0, STRIPE)])
    pltpu.sync_copy(ones_h, ones_v)
    _load_slabs(ei, padw, wid, src_v, dst_v)
    plsc.subcore_barrier()

    def body(j, carry):
        pltpu.async_copy(ones_v, dout_sh.at[src_v.at[j]], sem, add=True)
        pltpu.async_copy(ones_v, din_sh.at[dst_v.at[j]], sem, add=True)
        return carry

    lax.fori_loop(0, RPT, body, 0)

    def drain(j, carry):
        pltpu.make_async_copy(ones_h, ones_v, sem).wait()
        pltpu.make_async_copy(ones_h, ones_v, sem).wait()
        return carry

    lax.fori_loop(0, RPT, drain, 0)
    plsc.subcore_barrier()

    @pl.when(cid == 0)
    def _():
        pltpu.sync_copy(dout_sh.at[pl.ds(r0, STRIPE)], ds0.at[pl.ds(r0, STRIPE)])
        pltpu.sync_copy(din_sh.at[pl.ds(r0, STRIPE)], dd0.at[pl.ds(r0, STRIPE)])

    @pl.when(cid == 1)
    def _():
        pltpu.sync_copy(dout_sh.at[pl.ds(r0, STRIPE)], ds1.at[pl.ds(r0, STRIPE)])
        pltpu.sync_copy(din_sh.at[pl.ds(r0, STRIPE)], dd1.at[pl.ds(r0, STRIPE)])


K = 4
NIT = RPT // (2 * K)


def _agg_pipeline(table, acc_sh, src_v, dst_v, buf_a, buf_b, sem_g, sem_s):
    for t in range(K):
        pltpu.async_copy(table.at[src_v.at[t]], buf_a.at[t], sem_g)

    def body(gg, carry):
        b0 = gg * 2 * K
        b1 = b0 + K
        for t in range(K):
            pltpu.make_async_copy(table.at[pl.ds(0, 128)], buf_a.at[t],
                                  sem_g).wait()
        s_a = [
            pltpu.async_copy(buf_a.at[t], acc_sh.at[dst_v.at[b0 + t]],
                             sem_s, add=True)
            for t in range(K)
        ]
        g_b = [
            pltpu.async_copy(table.at[src_v.at[b1 + t]], buf_b.at[t], sem_g)
            for t in range(K)
        ]
        for d in s_a:
            d.wait()
        for d in g_b:
            d.wait()
        s_b = [
            pltpu.async_copy(buf_b.at[t], acc_sh.at[dst_v.at[b1 + t]],
                             sem_s, add=True)
            for t in range(K)
        ]

        @pl.when(gg < NIT - 1)
        def _():
            for t in range(K):
                pltpu.async_copy(table.at[src_v.at[b1 + K + t]],
                                 buf_a.at[t], sem_g)

        for d in s_b:
            d.wait()
        return carry

    lax.fori_loop(0, NIT, body, 0)


def _make_sc_agg():
    feat = D_OUT

    @functools.partial(
        pl.kernel,
        mesh=_MESH,
        compiler_params=pltpu.CompilerParams(use_tc_tiling_on_sc=False),
        out_type=[jax.ShapeDtypeStruct((NP, feat), jnp.float32) for _ in range(2)],
        scratch_types=[
            pltpu.VMEM((RPT, 128), jnp.int32),
            pltpu.VMEM((RPT, 128), jnp.int32),
            pltpu.VMEM((K, 128, feat), jnp.float32),
            pltpu.VMEM((K, 128, feat), jnp.float32),
            pltpu.VMEM_SHARED((NP, feat), jnp.float32),
            pltpu.SemaphoreType.DMA,
            pltpu.SemaphoreType.DMA,
        ],
    )
    def agg(table, ei, padw, zeros_h, p0, p1,
            src_v, dst_v, buf_a, buf_b, acc_sh, sem_g, sem_s):
        cid = lax.axis_index("c")
        sid = lax.axis_index("s")
        wid = sid * NC + cid
        r0 = sid * STRIPE
        pltpu.sync_copy(zeros_h, acc_sh.at[pl.ds(r0, STRIPE)])
        _load_slabs(ei, padw, wid, src_v, dst_v)
        plsc.subcore_barrier()
        _agg_pipeline(table, acc_sh, src_v, dst_v, buf_a, buf_b, sem_g, sem_s)
        plsc.subcore_barrier()

        @pl.when(cid == 0)
        def _():
            pltpu.sync_copy(acc_sh.at[pl.ds(r0, STRIPE)], p0.at[pl.ds(r0, STRIPE)])

        @pl.when(cid == 1)
        def _():
            pltpu.sync_copy(acc_sh.at[pl.ds(r0, STRIPE)], p1.at[pl.ds(r0, STRIPE)])

    return agg


_sc_agg64 = _make_sc_agg()


@functools.partial(
    pl.kernel,
    mesh=_MESH,
    compiler_params=pltpu.CompilerParams(use_tc_tiling_on_sc=False),
    out_type=[jax.ShapeDtypeStruct((NP, D_OUT), jnp.float32) for _ in range(2)],
    scratch_types=[
        pltpu.VMEM((RPT, 128), jnp.int32),
        pltpu.VMEM((RPT, 128), jnp.int32),
        pltpu.VMEM((K, 128, D_OUT), jnp.float32),
        pltpu.VMEM((K, 128, D_OUT), jnp.float32),
        pltpu.VMEM_SHARED((NP, D_OUT), jnp.float32),
        pltpu.SemaphoreType.DMA,
        pltpu.SemaphoreType.DMA,
    ],
)
def _sc_agg_dual(table_l, table_r, ei, padw, zeros_h, p_l, p_r,
                 src_v, dst_v, buf_a, buf_b, acc_sh, sem_g, sem_s):
    cid = lax.axis_index("c")
    sid = lax.axis_index("s")
    r0 = sid * STRIPE

    def half(table, pout):
        pltpu.sync_copy(zeros_h, acc_sh.at[pl.ds(r0, STRIPE)])
        plsc.subcore_barrier()
        for p in range(2):
            _load_slabs(ei, padw, sid * 2 + p, src_v, dst_v)
            _agg_pipeline(table, acc_sh, src_v, dst_v, buf_a, buf_b,
                          sem_g, sem_s)
        plsc.subcore_barrier()
        pltpu.sync_copy(acc_sh.at[pl.ds(r0, STRIPE)], pout.at[pl.ds(r0, STRIPE)])

    @pl.when(cid == 0)
    def _():
        half(table_l, p_l)

    @pl.when(cid == 1)
    def _():
        half(table_r, p_r)


BR = 2048
BG = BR // 128
NB = NP // BR


def _norm4(a_ref, b_ref):
    return lax.rsqrt(jnp.maximum(a_ref[0] + b_ref[0], 1.0))


def _mm1_body(x_ref, w1_ref, yl_ref, yr_ref):
    y = jnp.dot(x_ref[...], w1_ref[...], preferred_element_type=jnp.float32)
    yl_ref[...] = y[:, :64]
    yr_ref[...] = y[:, 64:]


_mm1 = pl.pallas_call(
    _mm1_body,
    grid=(NB,),
    in_specs=[
        pl.BlockSpec((BR, D_IN), lambda i: (i, 0)),
        pl.BlockSpec((D_IN, D_HID), lambda i: (0, 0)),
    ],
    out_specs=[
        pl.BlockSpec((BR, 64), lambda i: (i, 0)),
        pl.BlockSpec((BR, 64), lambda i: (i, 0)),
    ],
    out_shape=[
        jax.ShapeDtypeStruct((NP, 64), jnp.float32),
        jax.ShapeDtypeStruct((NP, 64), jnp.float32),
    ],
)


def _scale_body(ul_ref, ur_ref, ds0_ref, ds1_ref, yl_ref, yr_ref):
    ns = _norm4(ds0_ref, ds1_ref)
    ns3 = ns[:, :, None]
    yl_ref[...] = jnp.reshape(
        jnp.reshape(ul_ref[...], (BG, 128, 64)) * ns3, (BR, 64))
    yr_ref[...] = jnp.reshape(
        jnp.reshape(ur_ref[...], (BG, 128, 64)) * ns3, (BR, 64))


_scale = pl.pallas_call(
    _scale_body,
    grid=(NB,),
    in_specs=[
        pl.BlockSpec((BR, 64), lambda i: (i, 0)),
        pl.BlockSpec((BR, 64), lambda i: (i, 0)),
        pl.BlockSpec((1, BG, 128), lambda i: (i, 0, 0)),
        pl.BlockSpec((1, BG, 128), lambda i: (i, 0, 0)),
    ],
    out_specs=[
        pl.BlockSpec((BR, 64), lambda i: (i, 0)),
        pl.BlockSpec((BR, 64), lambda i: (i, 0)),
    ],
    out_shape=[
        jax.ShapeDtypeStruct((NP, 64), jnp.float32),
        jax.ShapeDtypeStruct((NP, 64), jnp.float32),
    ],
)


def _mid_body(pl_ref, pr_ref, dd0_ref, dd1_ref,
              ds0_ref, ds1_ref, b1_ref, w2_ref, y_ref):
    nd = _norm4(dd0_ref, dd1_ref)
    ns = _norm4(ds0_ref, ds1_ref)
    agg2 = jnp.concatenate([pl_ref[...], pr_ref[...]], axis=-1)
    agg = jnp.reshape(agg2, (BG, 128, D_HID))
    h = jnp.maximum(agg * nd[:, :, None] + b1_ref[...][None, None, :], 0.0)
    hs = jnp.reshape(h * ns[:, :, None], (BR, D_HID))
    y_ref[...] = jnp.dot(hs, w2_ref[...], preferred_element_type=jnp.float32)


_mid = pl.pallas_call(
    _mid_body,
    grid=(NB,),
    in_specs=[
        pl.BlockSpec((BR, 64), lambda i: (i, 0)),
        pl.BlockSpec((BR, 64), lambda i: (i, 0)),
        pl.BlockSpec((1, BG, 128), lambda i: (i, 0, 0)),
        pl.BlockSpec((1, BG, 128), lambda i: (i, 0, 0)),
        pl.BlockSpec((1, BG, 128), lambda i: (i, 0, 0)),
        pl.BlockSpec((1, BG, 128), lambda i: (i, 0, 0)),
        pl.BlockSpec((D_HID,), lambda i: (0,)),
        pl.BlockSpec((D_HID, D_OUT), lambda i: (0, 0)),
    ],
    out_specs=pl.BlockSpec((BR, D_OUT), lambda i: (i, 0)),
    out_shape=jax.ShapeDtypeStruct((NP, D_OUT), jnp.float32),
)


def _final_body(p0_ref, p1_ref, dd0_ref, dd1_ref, b2_ref, o_ref):
    nd = _norm4(dd0_ref, dd1_ref)
    agg = jnp.reshape(p0_ref[...] + p1_ref[...], (BG, 128, D_OUT))
    z = agg * nd[:, :, None] + b2_ref[...][None, None, :]
    z = z - jnp.max(z, axis=-1, keepdims=True)
    ez = jnp.exp(z)
    sm = ez / jnp.sum(ez, axis=-1, keepdims=True)
    o_ref[...] = jnp.reshape(sm, (BR, D_OUT))


_final = pl.pallas_call(
    _final_body,
    grid=(NB,),
    in_specs=[
        pl.BlockSpec((BR, D_OUT), lambda i: (i, 0)),
        pl.BlockSpec((BR, D_OUT), lambda i: (i, 0)),
        pl.BlockSpec((1, BG, 128), lambda i: (i, 0, 0)),
        pl.BlockSpec((1, BG, 128), lambda i: (i, 0, 0)),
        pl.BlockSpec((D_OUT,), lambda i: (0,)),
    ],
    out_specs=pl.BlockSpec((BR, D_OUT), lambda i: (i, 0)),
    out_shape=jax.ShapeDtypeStruct((N, D_OUT), jnp.float32),
)


def kernel(x, edge_index, W1, b1, W2, b2):
    ei3 = edge_index.astype(jnp.int32).reshape(2, NROW, 128)
    padw = (jnp.arange(RPAD * 128, dtype=jnp.int32) % (NP - N) + N).reshape(
        RPAD, 128)

    ones1 = jnp.ones((128,), jnp.float32)
    zeros1 = jnp.zeros((STRIPE,), jnp.float32)
    zO = jnp.zeros((STRIPE, D_OUT), jnp.float32)

    ds0, ds1, dd0, dd1 = _sc_degrees(ei3, padw, ones1, zeros1)
    ds0 = ds0.reshape(NB, BG, 128)
    ds1 = ds1.reshape(NB, BG, 128)
    dd0 = dd0.reshape(NB, BG, 128)
    dd1 = dd1.reshape(NB, BG, 128)

    ul, ur = _mm1(x, W1)
    y1l, y1r = _scale(ul, ur, ds0, ds1)
    al, ar = _sc_agg_dual(y1l, y1r, ei3, padw, zO)
    y2 = _mid(al, ar, dd0, dd1, ds0, ds1, b1, W2)
    g0, g1 = _sc_agg64(y2, ei3, padw, zO)
    return _final(g0, g1, dd0, dd1, b2)
